# baseline (speedup 1.0000x reference)
import functools

import jax
import jax.numpy as jnp
from jax.experimental import pallas as pl

_MB = 8
_LO1, _HI1 = 1.0, 2.25
_LO2, _HI2 = 2.25, 9.0
_K1, _K2 = 16, 32
_CH1 = 64
_NEG = -3.0e38


def _cumsum_lanes(x):
    n = x.shape[1]
    sh = 1
    while sh < n:
        shifted = jnp.pad(x, ((0, 0), (sh, 0)))[:, :n]
        x = x + shifted
        sh *= 2
    return x


def _mm_body(xt_ref, cen_ref, wt_ref, w_ref, b_ref, at_ref, c_ref):
    wt = wt_ref[...]
    at_ref[0] = jnp.dot(wt, xt_ref[0], preferred_element_type=jnp.float32) + b_ref[...]
    c_ref[0] = jnp.dot(cen_ref[0], w_ref[0:3, :], preferred_element_type=jnp.float32)


def _ballmax_body(d_ref, at_ref, c_ref, wagg_ref, bagg_ref, o_ref):
    d = d_ref[0]
    ge1 = d >= _LO1
    ge2 = d >= _HI1
    lt9 = d < _HI2
    pred1 = ge1 & (~ge2)
    pred2 = ge2 & lt9
    cs1 = _cumsum_lanes(pred1.astype(jnp.int32))
    cs2 = _cumsum_lanes(pred2.astype(jnp.int32))
    m1 = pred1 & (cs1 <= _K1)
    m2 = pred2 & (cs2 <= _K2)
    at = at_ref[0]
    a1 = at[:_CH1]
    a2 = at[_CH1:]
    mxt = jnp.full((at.shape[0], _MB), _NEG, jnp.float32)
    lane = jax.lax.broadcasted_iota(jnp.int32, (at.shape[0], _MB), 1)
    for m in range(_MB):
        s1 = jnp.max(jnp.where(m1[m:m + 1, :], a1, _NEG),
                     axis=1, keepdims=True)
        s2 = jnp.max(jnp.where(m2[m:m + 1, :], a2, _NEG),
                     axis=1, keepdims=True)
        col = jnp.concatenate([s1, s2], axis=0)
        mxt = jnp.where(lane == m, col, mxt)
    mx = jnp.transpose(mxt, (1, 0))
    f = jax.nn.relu(mx - c_ref[0])
    o_ref[0] = jax.nn.relu(
        jnp.dot(f, wagg_ref[...], preferred_element_type=jnp.float32)
        + bagg_ref[...])


def kernel(positions, features, centers, distances, W0, b0, W1, b1, Wagg, bagg):
    B, N, D = positions.shape
    M = centers.shape[1]
    XT = jnp.concatenate([positions.transpose(0, 2, 1),
                          features.transpose(0, 2, 1)], axis=1)
    W = jnp.concatenate([W0, W1], axis=1)
    WT = W.T
    bcatT = jnp.concatenate([b0, b1])[:, None]
    CH = WT.shape[0]
    F = XT.shape[1]

    AT, Cc = pl.pallas_call(
        _mm_body,
        grid=(B,),
        in_specs=[
            pl.BlockSpec((1, F, N), lambda b: (b, 0, 0)),
            pl.BlockSpec((1, M, D), lambda b: (b, 0, 0)),
            pl.BlockSpec((CH, F), lambda b: (0, 0)),
            pl.BlockSpec((F, CH), lambda b: (0, 0)),
            pl.BlockSpec((CH, 1), lambda b: (0, 0)),
        ],
        out_specs=[
            pl.BlockSpec((1, CH, N), lambda b: (b, 0, 0)),
            pl.BlockSpec((1, M, CH), lambda b: (b, 0, 0)),
        ],
        out_shape=[
            jax.ShapeDtypeStruct((B, CH, N), jnp.float32),
            jax.ShapeDtypeStruct((B, M, CH), jnp.float32),
        ],
    )(XT, centers, WT, W, bcatT)

    OC = Wagg.shape[1]
    out = pl.pallas_call(
        _ballmax_body,
        grid=(B, M // _MB),
        in_specs=[
            pl.BlockSpec((1, _MB, N), lambda b, mi: (b, mi, 0)),
            pl.BlockSpec((1, CH, N), lambda b, mi: (b, 0, 0)),
            pl.BlockSpec((1, _MB, CH), lambda b, mi: (b, mi, 0)),
            pl.BlockSpec((CH, OC), lambda b, mi: (0, 0)),
            pl.BlockSpec((1, OC), lambda b, mi: (0, 0)),
        ],
        out_specs=pl.BlockSpec((1, _MB, OC), lambda b, mi: (b, mi, 0)),
        out_shape=jax.ShapeDtypeStruct((B, M, OC), jnp.float32),
    )(distances, AT, Cc, Wagg, bagg[None])
    return out


# speedup vs baseline: 25.7164x; 25.7164x over previous
import functools

import jax
import jax.numpy as jnp
from jax import lax
from jax.experimental import pallas as pl
from jax.experimental.pallas import tpu as pltpu
from jax.experimental.pallas import tpu_sc as plsc

_LO1, _HI1 = 1.0, 2.25
_LO2, _HI2 = 2.25, 9.0
_K1, _K2 = 16, 32
_CH1, _CH2 = 64, 128
_CH = _CH1 + _CH2
_NEG = -3.0e38
_MB2 = 512


def _mm_body(x_ref, w_ref, b_ref, a1_ref, a2_ref):
    bi = pl.program_id(0)
    last = pl.num_programs(0) - 1

    @pl.when(bi != last)
    def _():
        a = jnp.dot(x_ref[0], w_ref[...],
                    preferred_element_type=jnp.float32) + b_ref[...]
        a1_ref[0] = jnp.concatenate([a[:, :_CH1], a[:, :_CH1]], axis=1)
        a2_ref[0] = a[:, _CH1:]

    @pl.when(bi == last)
    def _():
        a1_ref[0] = jnp.full(a1_ref.shape[1:], _NEG, jnp.float32)
        a2_ref[0] = jnp.full(a2_ref.shape[1:], _NEG, jnp.float32)


def _fin_body(mx_ref, cen_ref, w3_ref, wagg_ref, bagg_ref, o_ref):
    c = jnp.dot(cen_ref[0], w3_ref[...], preferred_element_type=jnp.float32)
    f = jax.nn.relu(mx_ref[0] - c)
    o_ref[0] = jax.nn.relu(
        jnp.dot(f, wagg_ref[...], preferred_element_type=jnp.float32)
        + bagg_ref[...])


def _sc_select_max(dist2, a1t, a2t, B, M, N):
    NC, NS, L = 2, 16, 16
    NW = NC * NS
    RPW = (B * M) // NW
    PAD = B * N
    NV = N // L
    mesh = plsc.VectorSubcoreMesh(core_axis_name="c", subcore_axis_name="s",
                                  num_cores=NC, num_subcores=NS)

    @functools.partial(
        pl.kernel, mesh=mesh,
        compiler_params=pltpu.CompilerParams(needs_layout_passes=False),
        out_type=jax.ShapeDtypeStruct((B * M, _CH), jnp.float32),
        scratch_types=[
            pltpu.VMEM((N,), jnp.float32),
            pltpu.VMEM((_K1,), jnp.int32),
            pltpu.VMEM((_K2,), jnp.int32),
            pltpu.VMEM((_K1, 2 * _CH1), jnp.float32),
            pltpu.VMEM((_K2, _CH2), jnp.float32),
            pltpu.VMEM((_CH,), jnp.float32),
            pltpu.SemaphoreType.DMA,
            pltpu.SemaphoreType.DMA,
        ],
    )
    def sck(dist_hbm, a1_hbm, a2_hbm, out_hbm,
            dbuf, idx1, idx2, rows1, rows2, obuf, sg1, sg2):
        wid = lax.axis_index("s") * NC + lax.axis_index("c")
        base = wid * RPW
        iota = lax.broadcasted_iota(jnp.int32, (L,), 0)
        padv = jnp.full((L,), PAD, jnp.int32)
        zero = jnp.zeros((L,), jnp.int32)

        def row_body(i, carry):
            g = base + i
            b = g // M
            pltpu.sync_copy(dist_hbm.at[g], dbuf)
            idx1[...] = padv
            idx2[pl.ds(0, L)] = padv
            idx2[pl.ds(L, L)] = padv
            boff = b * N

            @plsc.parallel_loop(0, NV, unroll=4, carry=(zero, zero))
            def _scan(j, c2):
                t1, t2 = c2
                d = dbuf[pl.ds(j * L, L)]
                ge1 = d >= _LO1
                ge2 = d >= _HI1
                lt9 = d < _HI2
                p1 = ge1 & (~ge2)
                p2 = ge2 & lt9
                i1 = p1.astype(jnp.int32)
                i2 = p2.astype(jnp.int32)
                r1 = t1 + plsc.cumsum(i1) - i1
                r2 = t2 + plsc.cumsum(i2) - i2
                k1 = p1 & (r1 < _K1)
                k2 = p2 & (r2 < _K2)
                lidx = iota + (j * L + boff)
                plsc.store_scatter(idx1, [r1], lidx, mask=k1)
                plsc.store_scatter(idx2, [r2], lidx, mask=k2)
                t1 = t1 + plsc.all_reduce_population_count(p1)
                t2 = t2 + plsc.all_reduce_population_count(p2)
                return t1, t2
            cp1 = pltpu.async_copy(a1_hbm.at[idx1], rows1, sg1)
            cp2 = pltpu.async_copy(a2_hbm.at[idx2], rows2, sg2)
            cp1.wait()
            cp2.wait()
            for c in range(_CH1 // L):
                acc = rows1[0, pl.ds(c * L, L)]
                for r in range(1, _K1):
                    acc = jnp.maximum(acc, rows1[r, pl.ds(c * L, L)])
                obuf[pl.ds(c * L, L)] = acc
            for c in range(_CH2 // L):
                acc = rows2[0, pl.ds(c * L, L)]
                for r in range(1, _K2):
                    acc = jnp.maximum(acc, rows2[r, pl.ds(c * L, L)])
                obuf[pl.ds(_CH1 + c * L, L)] = acc
            pltpu.sync_copy(obuf, out_hbm.at[g])
            return carry

        lax.fori_loop(0, RPW, row_body, 0)

    return sck(dist2, a1t, a2t)


def kernel(positions, features, centers, distances, W0, b0, W1, b1, Wagg, bagg):
    B, N, D = positions.shape
    M = centers.shape[1]
    X = jnp.concatenate([positions, features], axis=-1)
    W = jnp.concatenate([W0, W1], axis=1)
    bcat = jnp.concatenate([b0, b1])[None]
    F = X.shape[-1]

    A1, A2 = pl.pallas_call(
        _mm_body,
        grid=(B + 1,),
        in_specs=[
            pl.BlockSpec((1, N, F), lambda b: (jnp.minimum(b, B - 1), 0, 0)),
            pl.BlockSpec((F, _CH), lambda b: (0, 0)),
            pl.BlockSpec((1, _CH), lambda b: (0, 0)),
        ],
        out_specs=[
            pl.BlockSpec((1, N, 2 * _CH1), lambda b: (b, 0, 0)),
            pl.BlockSpec((1, N, _CH2), lambda b: (b, 0, 0)),
        ],
        out_shape=[
            jax.ShapeDtypeStruct((B + 1, N, 2 * _CH1), jnp.float32),
            jax.ShapeDtypeStruct((B + 1, N, _CH2), jnp.float32),
        ],
    )(X, W, bcat)

    mx = _sc_select_max(
        distances.reshape(B * M, N),
        A1.reshape((B + 1) * N, 2 * _CH1),
        A2.reshape((B + 1) * N, _CH2),
        B, M, N).reshape(B, M, _CH)

    OC = Wagg.shape[1]
    out = pl.pallas_call(
        _fin_body,
        grid=(B, M // _MB2),
        in_specs=[
            pl.BlockSpec((1, _MB2, _CH), lambda b, mi: (b, mi, 0)),
            pl.BlockSpec((1, _MB2, D), lambda b, mi: (b, mi, 0)),
            pl.BlockSpec((D, _CH), lambda b, mi: (0, 0)),
            pl.BlockSpec((_CH, OC), lambda b, mi: (0, 0)),
            pl.BlockSpec((1, OC), lambda b, mi: (0, 0)),
        ],
        out_specs=pl.BlockSpec((1, _MB2, OC), lambda b, mi: (b, mi, 0)),
        out_shape=jax.ShapeDtypeStruct((B, M, OC), jnp.float32),
    )(mx, centers, W[0:3], Wagg, bagg[None])
    return out


# speedup vs baseline: 27.7310x; 1.0783x over previous
import functools

import jax
import jax.numpy as jnp
from jax import lax
from jax.experimental import pallas as pl
from jax.experimental.pallas import tpu as pltpu
from jax.experimental.pallas import tpu_sc as plsc

_LO1, _HI1 = 1.0, 2.25
_LO2, _HI2 = 2.25, 9.0
_K1, _K2 = 16, 32
_CH1, _CH2 = 64, 128
_CH = _CH1 + _CH2
_NEG = -3.0e38
_MB2 = 512


def _mm_body(x_ref, w_ref, b_ref, a1_ref, a2_ref):
    bi = pl.program_id(0)
    last = pl.num_programs(0) - 1

    @pl.when(bi != last)
    def _():
        a = jnp.dot(x_ref[0], w_ref[...],
                    preferred_element_type=jnp.float32) + b_ref[...]
        a1_ref[0] = jnp.concatenate([a[:, :_CH1], a[:, :_CH1]], axis=1)
        a2_ref[0] = a[:, _CH1:]

    @pl.when(bi == last)
    def _():
        a1_ref[0] = jnp.full(a1_ref.shape[1:], _NEG, jnp.float32)
        a2_ref[0] = jnp.full(a2_ref.shape[1:], _NEG, jnp.float32)


def _fin_body(mx_ref, cen_ref, w3_ref, wagg_ref, bagg_ref, o_ref):
    c = jnp.dot(cen_ref[0], w3_ref[...], preferred_element_type=jnp.float32)
    f = jax.nn.relu(mx_ref[0] - c)
    o_ref[0] = jax.nn.relu(
        jnp.dot(f, wagg_ref[...], preferred_element_type=jnp.float32)
        + bagg_ref[...])


def _sc_select_max(dist2, a1t, a2t, B, M, N):
    NC, NS, L = 2, 16, 16
    NW = NC * NS
    RPW = (B * M) // NW
    PAD = B * N
    NV = N // L
    _CVS = 32
    mesh = plsc.VectorSubcoreMesh(core_axis_name="c", subcore_axis_name="s",
                                  num_cores=NC, num_subcores=NS)

    @functools.partial(
        pl.kernel, mesh=mesh,
        compiler_params=pltpu.CompilerParams(needs_layout_passes=False),
        out_type=jax.ShapeDtypeStruct((B * M, _CH), jnp.float32),
        scratch_types=[
            pltpu.VMEM((N,), jnp.float32),
            pltpu.VMEM((_K1,), jnp.int32),
            pltpu.VMEM((_K2,), jnp.int32),
            pltpu.VMEM((_K1, 2 * _CH1), jnp.float32),
            pltpu.VMEM((_K2, _CH2), jnp.float32),
            pltpu.VMEM((_CH,), jnp.float32),
            pltpu.SemaphoreType.DMA,
            pltpu.SemaphoreType.DMA,
        ],
    )
    def sck(dist_hbm, a1_hbm, a2_hbm, out_hbm,
            dbuf, idx1, idx2, rows1, rows2, obuf, sg1, sg2):
        wid = lax.axis_index("s") * NC + lax.axis_index("c")
        base = wid * RPW
        iota = lax.broadcasted_iota(jnp.int32, (L,), 0)
        padv = jnp.full((L,), PAD, jnp.int32)
        zero = jnp.zeros((L,), jnp.int32)

        def row_body(i, carry):
            g = base + i
            b = g // M
            pltpu.sync_copy(dist_hbm.at[g], dbuf)
            idx1[...] = padv
            idx2[pl.ds(0, L)] = padv
            idx2[pl.ds(L, L)] = padv
            boff = b * N

            def _full_chunk(c, tt):
                t1, t2 = tt

                @plsc.parallel_loop(0, _CVS, unroll=4, carry=(t1, t2))
                def _scan(jj, c2):
                    t1c, t2c = c2
                    j = c * _CVS + jj
                    d = dbuf[pl.ds(j * L, L)]
                    ge1 = d >= _LO1
                    ge2 = d >= _HI1
                    lt9 = d < _HI2
                    p1 = ge1 & (~ge2)
                    p2 = ge2 & lt9
                    i1 = p1.astype(jnp.int32)
                    i2 = p2.astype(jnp.int32)
                    r1 = t1c + plsc.cumsum(i1) - i1
                    r2 = t2c + plsc.cumsum(i2) - i2
                    k1 = p1 & (r1 < _K1)
                    k2 = p2 & (r2 < _K2)
                    lidx = iota + (j * L + boff)
                    plsc.store_scatter(idx1, [r1], lidx, mask=k1)
                    plsc.store_scatter(idx2, [r2], lidx, mask=k2)
                    t1c = t1c + plsc.all_reduce_population_count(p1)
                    t2c = t2c + plsc.all_reduce_population_count(p2)
                    return t1c, t2c

                return _scan

            def _s2_chunk(c, tt):
                t1, t2 = tt

                @plsc.parallel_loop(0, _CVS, unroll=4, carry=t2)
                def _scan(jj, t2c):
                    j = c * _CVS + jj
                    d = dbuf[pl.ds(j * L, L)]
                    p2 = (d >= _HI1) & (d < _HI2)
                    i2 = p2.astype(jnp.int32)
                    r2 = t2c + plsc.cumsum(i2) - i2
                    k2 = p2 & (r2 < _K2)
                    lidx = iota + (j * L + boff)
                    plsc.store_scatter(idx2, [r2], lidx, mask=k2)
                    return t2c + plsc.all_reduce_population_count(p2)

                return t1, _scan

            def chunk_body(c, tt):
                cnt1 = jnp.max(tt[0], axis=0)
                cnt2 = jnp.max(tt[1], axis=0)
                return lax.cond(
                    cnt1 >= _K1,
                    lambda tt2: lax.cond(cnt2 >= _K2,
                                         lambda tt3: tt3,
                                         lambda tt3: _s2_chunk(c, tt3),
                                         tt2),
                    lambda tt2: _full_chunk(c, tt2),
                    tt)

            lax.fori_loop(0, NV // _CVS, chunk_body, (zero, zero))
            cp1 = pltpu.async_copy(a1_hbm.at[idx1], rows1, sg1)
            cp2 = pltpu.async_copy(a2_hbm.at[idx2], rows2, sg2)
            cp1.wait()
            cp2.wait()
            for c in range(_CH1 // L):
                acc = rows1[0, pl.ds(c * L, L)]
                for r in range(1, _K1):
                    acc = jnp.maximum(acc, rows1[r, pl.ds(c * L, L)])
                obuf[pl.ds(c * L, L)] = acc
            for c in range(_CH2 // L):
                acc = rows2[0, pl.ds(c * L, L)]
                for r in range(1, _K2):
                    acc = jnp.maximum(acc, rows2[r, pl.ds(c * L, L)])
                obuf[pl.ds(_CH1 + c * L, L)] = acc
            pltpu.sync_copy(obuf, out_hbm.at[g])
            return carry

        lax.fori_loop(0, RPW, row_body, 0)

    return sck(dist2, a1t, a2t)


def kernel(positions, features, centers, distances, W0, b0, W1, b1, Wagg, bagg):
    B, N, D = positions.shape
    M = centers.shape[1]
    X = jnp.concatenate([positions, features], axis=-1)
    W = jnp.concatenate([W0, W1], axis=1)
    bcat = jnp.concatenate([b0, b1])[None]
    F = X.shape[-1]

    A1, A2 = pl.pallas_call(
        _mm_body,
        grid=(B + 1,),
        in_specs=[
            pl.BlockSpec((1, N, F), lambda b: (jnp.minimum(b, B - 1), 0, 0)),
            pl.BlockSpec((F, _CH), lambda b: (0, 0)),
            pl.BlockSpec((1, _CH), lambda b: (0, 0)),
        ],
        out_specs=[
            pl.BlockSpec((1, N, 2 * _CH1), lambda b: (b, 0, 0)),
            pl.BlockSpec((1, N, _CH2), lambda b: (b, 0, 0)),
        ],
        out_shape=[
            jax.ShapeDtypeStruct((B + 1, N, 2 * _CH1), jnp.float32),
            jax.ShapeDtypeStruct((B + 1, N, _CH2), jnp.float32),
        ],
    )(X, W, bcat)

    mx = _sc_select_max(
        distances.reshape(B * M, N),
        A1.reshape((B + 1) * N, 2 * _CH1),
        A2.reshape((B + 1) * N, _CH2),
        B, M, N).reshape(B, M, _CH)

    OC = Wagg.shape[1]
    out = pl.pallas_call(
        _fin_body,
        grid=(B, M // _MB2),
        in_specs=[
            pl.BlockSpec((1, _MB2, _CH), lambda b, mi: (b, mi, 0)),
            pl.BlockSpec((1, _MB2, D), lambda b, mi: (b, mi, 0)),
            pl.BlockSpec((D, _CH), lambda b, mi: (0, 0)),
            pl.BlockSpec((_CH, OC), lambda b, mi: (0, 0)),
            pl.BlockSpec((1, OC), lambda b, mi: (0, 0)),
        ],
        out_specs=pl.BlockSpec((1, _MB2, OC), lambda b, mi: (b, mi, 0)),
        out_shape=jax.ShapeDtypeStruct((B, M, OC), jnp.float32),
    )(mx, centers, W[0:3], Wagg, bagg[None])
    return out


# speedup vs baseline: 45.6280x; 1.6454x over previous
import functools

import jax
import jax.numpy as jnp
from jax import lax
from jax.experimental import pallas as pl
from jax.experimental.pallas import tpu as pltpu
from jax.experimental.pallas import tpu_sc as plsc

_LO1, _HI1 = 1.0, 2.25
_LO2, _HI2 = 2.25, 9.0
_K1, _K2 = 16, 32
_CH1, _CH2 = 64, 128
_CH = _CH1 + _CH2
_NEG = -3.0e38
_MB2 = 512


def _mm_body(x_ref, w_ref, b_ref, a1_ref, a2_ref):
    bi = pl.program_id(0)
    last = pl.num_programs(0) - 1

    @pl.when(bi != last)
    def _():
        a = jnp.dot(x_ref[0], w_ref[...],
                    preferred_element_type=jnp.float32) + b_ref[...]
        a1_ref[0] = jnp.concatenate([a[:, :_CH1], a[:, :_CH1]], axis=1)
        a2_ref[0] = a[:, _CH1:]

    @pl.when(bi == last)
    def _():
        a1_ref[0] = jnp.full(a1_ref.shape[1:], _NEG, jnp.float32)
        a2_ref[0] = jnp.full(a2_ref.shape[1:], _NEG, jnp.float32)


def _fin_body(mx_ref, cen_ref, w3_ref, wagg_ref, bagg_ref, o_ref):
    c = jnp.dot(cen_ref[0], w3_ref[...], preferred_element_type=jnp.float32)
    f = jax.nn.relu(mx_ref[0] - c)
    o_ref[0] = jax.nn.relu(
        jnp.dot(f, wagg_ref[...], preferred_element_type=jnp.float32)
        + bagg_ref[...])


def _sc_select_max(dist2, a1t, a2t, B, M, N):
    NC, NS, L = 2, 16, 16
    NW = NC * NS
    RPW = (B * M) // NW
    PAD = B * N
    NV = N // L
    _CVS = 32
    mesh = plsc.VectorSubcoreMesh(core_axis_name="c", subcore_axis_name="s",
                                  num_cores=NC, num_subcores=NS)

    @functools.partial(
        pl.kernel, mesh=mesh,
        compiler_params=pltpu.CompilerParams(needs_layout_passes=False),
        out_type=jax.ShapeDtypeStruct((B * M, _CH), jnp.float32),
        scratch_types=[
            pltpu.VMEM((N,), jnp.float32),
            pltpu.VMEM((N,), jnp.float32),
            pltpu.VMEM((_K1,), jnp.int32),
            pltpu.VMEM((_K1,), jnp.int32),
            pltpu.VMEM((_K2,), jnp.int32),
            pltpu.VMEM((_K2,), jnp.int32),
            pltpu.VMEM((_K1, 2 * _CH1), jnp.float32),
            pltpu.VMEM((_K1, 2 * _CH1), jnp.float32),
            pltpu.VMEM((_K2, _CH2), jnp.float32),
            pltpu.VMEM((_K2, _CH2), jnp.float32),
            pltpu.VMEM((2, _CH), jnp.float32),
            pltpu.SemaphoreType.DMA,
            pltpu.SemaphoreType.DMA,
            pltpu.SemaphoreType.DMA,
            pltpu.SemaphoreType.DMA,
            pltpu.SemaphoreType.DMA,
            pltpu.SemaphoreType.DMA,
        ],
    )
    def sck(dist_hbm, a1_hbm, a2_hbm, out_hbm,
            dbufA, dbufB, idx1A, idx1B, idx2A, idx2B,
            rows1A, rows1B, rows2A, rows2B, obuf2,
            sdA, sdB, g1A, g2A, g1B, g2B):
        wid = lax.axis_index("s") * NC + lax.axis_index("c")
        base = wid * RPW
        BM = B * M
        iota = lax.broadcasted_iota(jnp.int32, (L,), 0)
        padv = jnp.full((L,), PAD, jnp.int32)
        zero = jnp.zeros((L,), jnp.int32)

        def scan_row(g, dbuf, idx1, idx2):
            b = g // M
            boff = b * N
            idx1[...] = padv
            idx2[pl.ds(0, L)] = padv
            idx2[pl.ds(L, L)] = padv

            def _full_chunk(c, tt):
                t1, t2 = tt

                @plsc.parallel_loop(0, _CVS, unroll=4, carry=(t1, t2))
                def _scan(jj, c2):
                    t1c, t2c = c2
                    j = c * _CVS + jj
                    d = dbuf[pl.ds(j * L, L)]
                    ge1 = d >= _LO1
                    ge2 = d >= _HI1
                    lt9 = d < _HI2
                    p1 = ge1 & (~ge2)
                    p2 = ge2 & lt9
                    i1 = p1.astype(jnp.int32)
                    i2 = p2.astype(jnp.int32)
                    r1 = t1c + plsc.cumsum(i1) - i1
                    r2 = t2c + plsc.cumsum(i2) - i2
                    k1 = p1 & (r1 < _K1)
                    k2 = p2 & (r2 < _K2)
                    lidx = iota + (j * L + boff)
                    plsc.store_scatter(idx1, [r1], lidx, mask=k1)
                    plsc.store_scatter(idx2, [r2], lidx, mask=k2)
                    t1c = t1c + plsc.all_reduce_population_count(p1)
                    t2c = t2c + plsc.all_reduce_population_count(p2)
                    return t1c, t2c

                return _scan

            def _s2_chunk(c, tt):
                t1, t2 = tt

                @plsc.parallel_loop(0, _CVS, unroll=4, carry=t2)
                def _scan(jj, t2c):
                    j = c * _CVS + jj
                    d = dbuf[pl.ds(j * L, L)]
                    p2 = (d >= _HI1) & (d < _HI2)
                    i2 = p2.astype(jnp.int32)
                    r2 = t2c + plsc.cumsum(i2) - i2
                    k2 = p2 & (r2 < _K2)
                    lidx = iota + (j * L + boff)
                    plsc.store_scatter(idx2, [r2], lidx, mask=k2)
                    return t2c + plsc.all_reduce_population_count(p2)

                return t1, _scan

            def chunk_body(c, tt):
                cnt1 = jnp.max(tt[0], axis=0)
                cnt2 = jnp.max(tt[1], axis=0)
                return lax.cond(
                    cnt1 >= _K1,
                    lambda tt2: lax.cond(cnt2 >= _K2,
                                         lambda tt3: tt3,
                                         lambda tt3: _s2_chunk(c, tt3),
                                         tt2),
                    lambda tt2: _full_chunk(c, tt2),
                    tt)

            lax.fori_loop(0, NV // _CVS, chunk_body, (zero, zero))

        def max_row(rows1, rows2, slot):
            for c in range(_CH1 // L):
                acc = rows1[0, pl.ds(c * L, L)]
                for r in range(1, _K1):
                    acc = jnp.maximum(acc, rows1[r, pl.ds(c * L, L)])
                obuf2[slot, pl.ds(c * L, L)] = acc
            for c in range(_CH2 // L):
                acc = rows2[0, pl.ds(c * L, L)]
                for r in range(1, _K2):
                    acc = jnp.maximum(acc, rows2[r, pl.ds(c * L, L)])
                obuf2[slot, pl.ds(_CH1 + c * L, L)] = acc

        pltpu.async_copy(dist_hbm.at[base], dbufA, sdA)
        pltpu.async_copy(dist_hbm.at[base + 1], dbufB, sdB)

        def pair_body(p, carry):
            ga = base + 2 * p
            gb = ga + 1
            pltpu.make_async_copy(dist_hbm.at[ga], dbufA, sdA).wait()
            scan_row(ga, dbufA, idx1A, idx2A)
            cpA1 = pltpu.async_copy(a1_hbm.at[idx1A], rows1A, g1A)
            cpA2 = pltpu.async_copy(a2_hbm.at[idx2A], rows2A, g2A)
            pltpu.async_copy(dist_hbm.at[jnp.minimum(ga + 2, BM - 1)],
                             dbufA, sdA)
            pltpu.make_async_copy(dist_hbm.at[gb], dbufB, sdB).wait()
            scan_row(gb, dbufB, idx1B, idx2B)
            cpB1 = pltpu.async_copy(a1_hbm.at[idx1B], rows1B, g1B)
            cpB2 = pltpu.async_copy(a2_hbm.at[idx2B], rows2B, g2B)
            pltpu.async_copy(dist_hbm.at[jnp.minimum(gb + 2, BM - 1)],
                             dbufB, sdB)
            cpA1.wait()
            cpA2.wait()
            max_row(rows1A, rows2A, 0)
            cpB1.wait()
            cpB2.wait()
            max_row(rows1B, rows2B, 1)
            pltpu.sync_copy(obuf2, out_hbm.at[pl.ds(ga, 2)])
            return carry

        lax.fori_loop(0, RPW // 2, pair_body, 0)
        pltpu.make_async_copy(dist_hbm.at[jnp.minimum(base + RPW, BM - 1)],
                              dbufA, sdA).wait()
        pltpu.make_async_copy(dist_hbm.at[jnp.minimum(base + RPW + 1, BM - 1)],
                              dbufB, sdB).wait()

    return sck(dist2, a1t, a2t)


def kernel(positions, features, centers, distances, W0, b0, W1, b1, Wagg, bagg):
    B, N, D = positions.shape
    M = centers.shape[1]
    X = jnp.concatenate([positions, features], axis=-1)
    W = jnp.concatenate([W0, W1], axis=1)
    bcat = jnp.concatenate([b0, b1])[None]
    F = X.shape[-1]

    A1, A2 = pl.pallas_call(
        _mm_body,
        grid=(B + 1,),
        in_specs=[
            pl.BlockSpec((1, N, F), lambda b: (jnp.minimum(b, B - 1), 0, 0)),
            pl.BlockSpec((F, _CH), lambda b: (0, 0)),
            pl.BlockSpec((1, _CH), lambda b: (0, 0)),
        ],
        out_specs=[
            pl.BlockSpec((1, N, 2 * _CH1), lambda b: (b, 0, 0)),
            pl.BlockSpec((1, N, _CH2), lambda b: (b, 0, 0)),
        ],
        out_shape=[
            jax.ShapeDtypeStruct((B + 1, N, 2 * _CH1), jnp.float32),
            jax.ShapeDtypeStruct((B + 1, N, _CH2), jnp.float32),
        ],
    )(X, W, bcat)

    mx = _sc_select_max(
        distances.reshape(B * M, N),
        A1.reshape((B + 1) * N, 2 * _CH1),
        A2.reshape((B + 1) * N, _CH2),
        B, M, N).reshape(B, M, _CH)

    OC = Wagg.shape[1]
    out = pl.pallas_call(
        _fin_body,
        grid=(B, M // _MB2),
        in_specs=[
            pl.BlockSpec((1, _MB2, _CH), lambda b, mi: (b, mi, 0)),
            pl.BlockSpec((1, _MB2, D), lambda b, mi: (b, mi, 0)),
            pl.BlockSpec((D, _CH), lambda b, mi: (0, 0)),
            pl.BlockSpec((_CH, OC), lambda b, mi: (0, 0)),
            pl.BlockSpec((1, OC), lambda b, mi: (0, 0)),
        ],
        out_specs=pl.BlockSpec((1, _MB2, OC), lambda b, mi: (b, mi, 0)),
        out_shape=jax.ShapeDtypeStruct((B, M, OC), jnp.float32),
    )(mx, centers, W[0:3], Wagg, bagg[None])
    return out


# speedup vs baseline: 46.1849x; 1.0122x over previous
import functools

import jax
import jax.numpy as jnp
from jax import lax
from jax.experimental import pallas as pl
from jax.experimental.pallas import tpu as pltpu
from jax.experimental.pallas import tpu_sc as plsc

_LO1, _HI1 = 1.0, 2.25
_LO2, _HI2 = 2.25, 9.0
_K1, _K2 = 16, 32
_CH1, _CH2 = 64, 128
_CH = _CH1 + _CH2
_NEG = -3.0e38
_MB2 = 512


def _mm_body(x_ref, w_ref, b_ref, a1_ref, a2_ref):
    bi = pl.program_id(0)
    last = pl.num_programs(0) - 1

    @pl.when(bi != last)
    def _():
        a = jnp.dot(x_ref[0], w_ref[...],
                    preferred_element_type=jnp.float32) + b_ref[...]
        a1_ref[0] = jnp.concatenate([a[:, :_CH1], a[:, :_CH1]], axis=1)
        a2_ref[0] = a[:, _CH1:]

    @pl.when(bi == last)
    def _():
        a1_ref[0] = jnp.full(a1_ref.shape[1:], _NEG, jnp.float32)
        a2_ref[0] = jnp.full(a2_ref.shape[1:], _NEG, jnp.float32)


def _fin_body(mx_ref, cen_ref, w3_ref, wagg_ref, bagg_ref, o_ref):
    c = jnp.dot(cen_ref[0], w3_ref[...], preferred_element_type=jnp.float32)
    f = jax.nn.relu(mx_ref[0] - c)
    o_ref[0] = jax.nn.relu(
        jnp.dot(f, wagg_ref[...], preferred_element_type=jnp.float32)
        + bagg_ref[...])


def _sc_select_max(dist2, a1t, a2t, B, M, N):
    NC, NS, L = 2, 16, 16
    NW = NC * NS
    RPW = (B * M) // NW
    PAD = B * N
    NV = N // L
    _CVS = 32
    mesh = plsc.VectorSubcoreMesh(core_axis_name="c", subcore_axis_name="s",
                                  num_cores=NC, num_subcores=NS)

    @functools.partial(
        pl.kernel, mesh=mesh,
        compiler_params=pltpu.CompilerParams(needs_layout_passes=False),
        out_type=jax.ShapeDtypeStruct((B * M, _CH), jnp.float32),
        scratch_types=[
            pltpu.VMEM((N,), jnp.float32),
            pltpu.VMEM((N,), jnp.float32),
            pltpu.VMEM((_K1,), jnp.int32),
            pltpu.VMEM((_K1,), jnp.int32),
            pltpu.VMEM((_K2,), jnp.int32),
            pltpu.VMEM((_K2,), jnp.int32),
            pltpu.VMEM((_K1, 2 * _CH1), jnp.float32),
            pltpu.VMEM((_K1, 2 * _CH1), jnp.float32),
            pltpu.VMEM((_K2, _CH2), jnp.float32),
            pltpu.VMEM((_K2, _CH2), jnp.float32),
            pltpu.VMEM((2, _CH), jnp.float32),
            pltpu.SemaphoreType.DMA,
            pltpu.SemaphoreType.DMA,
            pltpu.SemaphoreType.DMA,
            pltpu.SemaphoreType.DMA,
            pltpu.SemaphoreType.DMA,
            pltpu.SemaphoreType.DMA,
            pltpu.SemaphoreType.DMA,
        ],
    )
    def sck(dist_hbm, a1_hbm, a2_hbm, out_hbm,
            dbufA, dbufB, idx1A, idx1B, idx2A, idx2B,
            rows1A, rows1B, rows2A, rows2B, obuf2,
            sdA, sdB, g1A, g2A, g1B, g2B, so):
        wid = lax.axis_index("s") * NC + lax.axis_index("c")
        base = wid * RPW
        BM = B * M
        iota = lax.broadcasted_iota(jnp.int32, (L,), 0)
        padv = jnp.full((L,), PAD, jnp.int32)
        zero = jnp.zeros((L,), jnp.int32)

        def scan_row(g, dbuf, idx1, idx2):
            b = g // M
            boff = b * N
            idx1[...] = padv
            idx2[pl.ds(0, L)] = padv
            idx2[pl.ds(L, L)] = padv

            lstep = jnp.full((L,), L, jnp.int32)
            cstep = jnp.full((L,), _CVS * L, jnp.int32)

            def _full_chunk(c, tt):
                t1, t2, lb = tt

                @plsc.parallel_loop(0, _CVS, unroll=4, carry=(t1, t2, lb))
                def _scan(jj, c2):
                    t1c, t2c, lidx = c2
                    j = c * _CVS + jj
                    d = dbuf[pl.ds(j * L, L)]
                    ge1 = d >= _LO1
                    ge2 = d >= _HI1
                    lt9 = d < _HI2
                    p1 = ge1 & (~ge2)
                    p2 = ge2 & lt9
                    i1 = p1.astype(jnp.int32)
                    i2 = p2.astype(jnp.int32)
                    r1 = t1c + plsc.cumsum(i1) - i1
                    r2 = t2c + plsc.cumsum(i2) - i2
                    k1 = p1 & (r1 < _K1)
                    k2 = p2 & (r2 < _K2)
                    plsc.store_scatter(idx1, [r1], lidx, mask=k1)
                    plsc.store_scatter(idx2, [r2], lidx, mask=k2)
                    t1c = t1c + plsc.all_reduce_population_count(p1)
                    t2c = t2c + plsc.all_reduce_population_count(p2)
                    return t1c, t2c, lidx + lstep

                return _scan

            def _s2_chunk(c, tt):
                t1, t2, lb = tt

                @plsc.parallel_loop(0, _CVS, unroll=8, carry=(t2, lb))
                def _scan(jj, c2):
                    t2c, lidx = c2
                    j = c * _CVS + jj
                    d = dbuf[pl.ds(j * L, L)]
                    p2 = (d >= _HI1) & (d < _HI2)
                    i2 = p2.astype(jnp.int32)
                    r2 = t2c + plsc.cumsum(i2) - i2
                    k2 = p2 & (r2 < _K2)
                    plsc.store_scatter(idx2, [r2], lidx, mask=k2)
                    return t2c + plsc.all_reduce_population_count(p2), lidx + lstep

                return t1, _scan[0], _scan[1]

            def chunk_body(c, tt):
                cnt1 = jnp.max(tt[0], axis=0)
                cnt2 = jnp.max(tt[1], axis=0)
                return lax.cond(
                    cnt1 >= _K1,
                    lambda tt2: lax.cond(
                        cnt2 >= _K2,
                        lambda tt3: (tt3[0], tt3[1], tt3[2] + cstep),
                        lambda tt3: _s2_chunk(c, tt3),
                        tt2),
                    lambda tt2: _full_chunk(c, tt2),
                    tt)

            lax.fori_loop(0, NV // _CVS, chunk_body,
                          (zero, zero, iota + boff))

        def max_row(rows1, rows2, slot):
            for c in range(_CH1 // L):
                acc = rows1[0, pl.ds(c * L, L)]
                for r in range(1, _K1):
                    acc = jnp.maximum(acc, rows1[r, pl.ds(c * L, L)])
                obuf2[slot, pl.ds(c * L, L)] = acc
            for c in range(_CH2 // L):
                acc = rows2[0, pl.ds(c * L, L)]
                for r in range(1, _K2):
                    acc = jnp.maximum(acc, rows2[r, pl.ds(c * L, L)])
                obuf2[slot, pl.ds(_CH1 + c * L, L)] = acc

        pltpu.async_copy(dist_hbm.at[base], dbufA, sdA)
        pltpu.async_copy(dist_hbm.at[base + 1], dbufB, sdB)

        def pair_body(p, carry):
            ga = base + 2 * p
            gb = ga + 1
            pltpu.make_async_copy(dist_hbm.at[ga], dbufA, sdA).wait()
            scan_row(ga, dbufA, idx1A, idx2A)
            cpA1 = pltpu.async_copy(a1_hbm.at[idx1A], rows1A, g1A)
            cpA2 = pltpu.async_copy(a2_hbm.at[idx2A], rows2A, g2A)
            pltpu.async_copy(dist_hbm.at[jnp.minimum(ga + 2, BM - 1)],
                             dbufA, sdA)
            pltpu.make_async_copy(dist_hbm.at[gb], dbufB, sdB).wait()
            scan_row(gb, dbufB, idx1B, idx2B)
            cpB1 = pltpu.async_copy(a1_hbm.at[idx1B], rows1B, g1B)
            cpB2 = pltpu.async_copy(a2_hbm.at[idx2B], rows2B, g2B)
            pltpu.async_copy(dist_hbm.at[jnp.minimum(gb + 2, BM - 1)],
                             dbufB, sdB)
            cpA1.wait()
            cpA2.wait()

            @pl.when(p > 0)
            def _():
                pltpu.make_async_copy(obuf2, out_hbm.at[pl.ds(ga - 2, 2)],
                                      so).wait()

            max_row(rows1A, rows2A, 0)
            cpB1.wait()
            cpB2.wait()
            max_row(rows1B, rows2B, 1)
            pltpu.async_copy(obuf2, out_hbm.at[pl.ds(ga, 2)], so)
            return carry

        lax.fori_loop(0, RPW // 2, pair_body, 0)
        pltpu.make_async_copy(obuf2, out_hbm.at[pl.ds(base + RPW - 2, 2)],
                              so).wait()
        pltpu.make_async_copy(dist_hbm.at[jnp.minimum(base + RPW, BM - 1)],
                              dbufA, sdA).wait()
        pltpu.make_async_copy(dist_hbm.at[jnp.minimum(base + RPW + 1, BM - 1)],
                              dbufB, sdB).wait()

    return sck(dist2, a1t, a2t)


def kernel(positions, features, centers, distances, W0, b0, W1, b1, Wagg, bagg):
    B, N, D = positions.shape
    M = centers.shape[1]
    X = jnp.concatenate([positions, features], axis=-1)
    W = jnp.concatenate([W0, W1], axis=1)
    bcat = jnp.concatenate([b0, b1])[None]
    F = X.shape[-1]

    A1, A2 = pl.pallas_call(
        _mm_body,
        grid=(B + 1,),
        in_specs=[
            pl.BlockSpec((1, N, F), lambda b: (jnp.minimum(b, B - 1), 0, 0)),
            pl.BlockSpec((F, _CH), lambda b: (0, 0)),
            pl.BlockSpec((1, _CH), lambda b: (0, 0)),
        ],
        out_specs=[
            pl.BlockSpec((1, N, 2 * _CH1), lambda b: (b, 0, 0)),
            pl.BlockSpec((1, N, _CH2), lambda b: (b, 0, 0)),
        ],
        out_shape=[
            jax.ShapeDtypeStruct((B + 1, N, 2 * _CH1), jnp.float32),
            jax.ShapeDtypeStruct((B + 1, N, _CH2), jnp.float32),
        ],
    )(X, W, bcat)

    mx = _sc_select_max(
        distances.reshape(B * M, N),
        A1.reshape((B + 1) * N, 2 * _CH1),
        A2.reshape((B + 1) * N, _CH2),
        B, M, N).reshape(B, M, _CH)

    OC = Wagg.shape[1]
    out = pl.pallas_call(
        _fin_body,
        grid=(B, M // _MB2),
        in_specs=[
            pl.BlockSpec((1, _MB2, _CH), lambda b, mi: (b, mi, 0)),
            pl.BlockSpec((1, _MB2, D), lambda b, mi: (b, mi, 0)),
            pl.BlockSpec((D, _CH), lambda b, mi: (0, 0)),
            pl.BlockSpec((_CH, OC), lambda b, mi: (0, 0)),
            pl.BlockSpec((1, OC), lambda b, mi: (0, 0)),
        ],
        out_specs=pl.BlockSpec((1, _MB2, OC), lambda b, mi: (b, mi, 0)),
        out_shape=jax.ShapeDtypeStruct((B, M, OC), jnp.float32),
    )(mx, centers, W[0:3], Wagg, bagg[None])
    return out


# speedup vs baseline: 47.2019x; 1.0220x over previous
import functools

import jax
import jax.numpy as jnp
from jax import lax
from jax.experimental import pallas as pl
from jax.experimental.pallas import tpu as pltpu
from jax.experimental.pallas import tpu_sc as plsc

_LO1, _HI1 = 1.0, 2.25
_LO2, _HI2 = 2.25, 9.0
_K1, _K2 = 16, 32
_CH1, _CH2 = 64, 128
_CH = _CH1 + _CH2
_NEG = -3.0e38
_MB2 = 512


def _mm_body(x_ref, w_ref, b_ref, a1_ref, a2_ref):
    bi = pl.program_id(0)
    last = pl.num_programs(0) - 1

    @pl.when(bi != last)
    def _():
        a = jnp.dot(x_ref[0], w_ref[...],
                    preferred_element_type=jnp.float32) + b_ref[...]
        a1_ref[0] = jnp.concatenate([a[:, :_CH1], a[:, :_CH1]], axis=1)
        a2_ref[0] = a[:, _CH1:]

    @pl.when(bi == last)
    def _():
        a1_ref[0] = jnp.full(a1_ref.shape[1:], _NEG, jnp.float32)
        a2_ref[0] = jnp.full(a2_ref.shape[1:], _NEG, jnp.float32)


def _fin_body(mx_ref, cen_ref, w3_ref, wagg_ref, bagg_ref, o_ref):
    c = jnp.dot(cen_ref[0], w3_ref[...], preferred_element_type=jnp.float32)
    f = jax.nn.relu(mx_ref[0] - c)
    o_ref[0] = jax.nn.relu(
        jnp.dot(f, wagg_ref[...], preferred_element_type=jnp.float32)
        + bagg_ref[...])


def _sc_select_max(dist2, a1t, a2t, B, M, N):
    NC, NS, L = 2, 16, 16
    NW = NC * NS
    RPW = (B * M) // NW
    PAD = B * N
    NV = N // L
    _CVS = 32
    mesh = plsc.VectorSubcoreMesh(core_axis_name="c", subcore_axis_name="s",
                                  num_cores=NC, num_subcores=NS)

    @functools.partial(
        pl.kernel, mesh=mesh,
        compiler_params=pltpu.CompilerParams(needs_layout_passes=False),
        out_type=jax.ShapeDtypeStruct((B * M, _CH), jnp.float32),
        scratch_types=[
            pltpu.VMEM((N,), jnp.float32),
            pltpu.VMEM((N,), jnp.float32),
            pltpu.VMEM((_K1,), jnp.int32),
            pltpu.VMEM((_K1,), jnp.int32),
            pltpu.VMEM((_K2,), jnp.int32),
            pltpu.VMEM((_K2,), jnp.int32),
            pltpu.VMEM((_K1, 2 * _CH1), jnp.float32),
            pltpu.VMEM((_K1, 2 * _CH1), jnp.float32),
            pltpu.VMEM((_K2, _CH2), jnp.float32),
            pltpu.VMEM((_K2, _CH2), jnp.float32),
            pltpu.VMEM((2, _CH), jnp.float32),
            pltpu.SemaphoreType.DMA,
            pltpu.SemaphoreType.DMA,
            pltpu.SemaphoreType.DMA,
            pltpu.SemaphoreType.DMA,
            pltpu.SemaphoreType.DMA,
            pltpu.SemaphoreType.DMA,
            pltpu.SemaphoreType.DMA,
        ],
    )
    def sck(dist_hbm, a1_hbm, a2_hbm, out_hbm,
            dbufA, dbufB, idx1A, idx1B, idx2A, idx2B,
            rows1A, rows1B, rows2A, rows2B, obuf2,
            sdA, sdB, g1A, g2A, g1B, g2B, so):
        wid = lax.axis_index("s") * NC + lax.axis_index("c")
        base = wid * RPW
        BM = B * M
        iota = lax.broadcasted_iota(jnp.int32, (L,), 0)
        padv = jnp.full((L,), PAD, jnp.int32)
        zero = jnp.zeros((L,), jnp.int32)

        def scan_row(g, dbuf, idx1, idx2):
            b = g // M
            boff = b * N
            idx1[...] = padv
            idx2[pl.ds(0, L)] = padv
            idx2[pl.ds(L, L)] = padv

            lstep = jnp.full((L,), L, jnp.int32)
            cstep = jnp.full((L,), _CVS * L, jnp.int32)

            def _full_chunk(c, tt):
                t1, t2, lb = tt

                @plsc.parallel_loop(0, _CVS, unroll=4, carry=(t1, t2, lb))
                def _scan(jj, c2):
                    t1c, t2c, lidx = c2
                    j = c * _CVS + jj
                    d = dbuf[pl.ds(j * L, L)]
                    ge1 = d >= _LO1
                    ge2 = d >= _HI1
                    lt9 = d < _HI2
                    p1 = ge1 & (~ge2)
                    p2 = ge2 & lt9
                    i1 = p1.astype(jnp.int32)
                    i2 = p2.astype(jnp.int32)
                    r1 = t1c + plsc.cumsum(i1) - i1
                    r2 = t2c + plsc.cumsum(i2) - i2
                    k1 = p1 & (r1 < _K1)
                    k2 = p2 & (r2 < _K2)
                    plsc.store_scatter(idx1, [r1], lidx, mask=k1)
                    plsc.store_scatter(idx2, [r2], lidx, mask=k2)
                    t1c = t1c + plsc.all_reduce_population_count(p1)
                    t2c = t2c + plsc.all_reduce_population_count(p2)
                    return t1c, t2c, lidx + lstep

                return _scan

            def _s2_chunk(c, tt):
                t1, t2, lb = tt

                @plsc.parallel_loop(0, _CVS, unroll=8, carry=(t2, lb))
                def _scan(jj, c2):
                    t2c, lidx = c2
                    j = c * _CVS + jj
                    d = dbuf[pl.ds(j * L, L)]
                    p2 = (d >= _HI1) & (d < _HI2)
                    i2 = p2.astype(jnp.int32)
                    r2 = t2c + plsc.cumsum(i2) - i2
                    k2 = p2 & (r2 < _K2)
                    plsc.store_scatter(idx2, [r2], lidx, mask=k2)
                    return t2c + plsc.all_reduce_population_count(p2), lidx + lstep

                return t1, _scan[0], _scan[1]

            def chunk_body(c, tt):
                s = jnp.max(tt[0] * 65536 + tt[1], axis=0)
                cnt1 = s // 65536
                cnt2 = s - cnt1 * 65536
                return lax.cond(
                    cnt1 >= _K1,
                    lambda tt2: lax.cond(
                        cnt2 >= _K2,
                        lambda tt3: (tt3[0], tt3[1], tt3[2] + cstep),
                        lambda tt3: _s2_chunk(c, tt3),
                        tt2),
                    lambda tt2: _full_chunk(c, tt2),
                    tt)

            lax.fori_loop(0, NV // _CVS, chunk_body,
                          (zero, zero, iota + boff))

        def max_row(rows1, rows2, slot):
            for c in range(_CH1 // L):
                acc = rows1[0, pl.ds(c * L, L)]
                for r in range(1, _K1):
                    acc = jnp.maximum(acc, rows1[r, pl.ds(c * L, L)])
                obuf2[slot, pl.ds(c * L, L)] = acc
            for c in range(_CH2 // L):
                acc = rows2[0, pl.ds(c * L, L)]
                for r in range(1, _K2):
                    acc = jnp.maximum(acc, rows2[r, pl.ds(c * L, L)])
                obuf2[slot, pl.ds(_CH1 + c * L, L)] = acc

        pltpu.async_copy(dist_hbm.at[base], dbufA, sdA)
        pltpu.async_copy(dist_hbm.at[base + 1], dbufB, sdB)

        def pair_body(p, carry):
            ga = base + 2 * p
            gb = ga + 1
            pltpu.make_async_copy(dist_hbm.at[ga], dbufA, sdA).wait()
            scan_row(ga, dbufA, idx1A, idx2A)
            cpA1 = pltpu.async_copy(a1_hbm.at[idx1A], rows1A, g1A)
            cpA2 = pltpu.async_copy(a2_hbm.at[idx2A], rows2A, g2A)
            pltpu.async_copy(dist_hbm.at[jnp.minimum(ga + 2, BM - 1)],
                             dbufA, sdA)
            pltpu.make_async_copy(dist_hbm.at[gb], dbufB, sdB).wait()
            scan_row(gb, dbufB, idx1B, idx2B)
            cpB1 = pltpu.async_copy(a1_hbm.at[idx1B], rows1B, g1B)
            cpB2 = pltpu.async_copy(a2_hbm.at[idx2B], rows2B, g2B)
            pltpu.async_copy(dist_hbm.at[jnp.minimum(gb + 2, BM - 1)],
                             dbufB, sdB)
            cpA1.wait()
            cpA2.wait()

            @pl.when(p > 0)
            def _():
                pltpu.make_async_copy(obuf2, out_hbm.at[pl.ds(ga - 2, 2)],
                                      so).wait()

            max_row(rows1A, rows2A, 0)
            cpB1.wait()
            cpB2.wait()
            max_row(rows1B, rows2B, 1)
            pltpu.async_copy(obuf2, out_hbm.at[pl.ds(ga, 2)], so)
            return carry

        lax.fori_loop(0, RPW // 2, pair_body, 0)
        pltpu.make_async_copy(obuf2, out_hbm.at[pl.ds(base + RPW - 2, 2)],
                              so).wait()
        pltpu.make_async_copy(dist_hbm.at[jnp.minimum(base + RPW, BM - 1)],
                              dbufA, sdA).wait()
        pltpu.make_async_copy(dist_hbm.at[jnp.minimum(base + RPW + 1, BM - 1)],
                              dbufB, sdB).wait()

    return sck(dist2, a1t, a2t)


def kernel(positions, features, centers, distances, W0, b0, W1, b1, Wagg, bagg):
    B, N, D = positions.shape
    M = centers.shape[1]
    X = jnp.concatenate([positions, features], axis=-1)
    W = jnp.concatenate([W0, W1], axis=1)
    bcat = jnp.concatenate([b0, b1])[None]
    F = X.shape[-1]

    A1, A2 = pl.pallas_call(
        _mm_body,
        grid=(B + 1,),
        in_specs=[
            pl.BlockSpec((1, N, F), lambda b: (jnp.minimum(b, B - 1), 0, 0)),
            pl.BlockSpec((F, _CH), lambda b: (0, 0)),
            pl.BlockSpec((1, _CH), lambda b: (0, 0)),
        ],
        out_specs=[
            pl.BlockSpec((1, N, 2 * _CH1), lambda b: (b, 0, 0)),
            pl.BlockSpec((1, N, _CH2), lambda b: (b, 0, 0)),
        ],
        out_shape=[
            jax.ShapeDtypeStruct((B + 1, N, 2 * _CH1), jnp.float32),
            jax.ShapeDtypeStruct((B + 1, N, _CH2), jnp.float32),
        ],
    )(X, W, bcat)

    mx = _sc_select_max(
        distances.reshape(B * M, N),
        A1.reshape((B + 1) * N, 2 * _CH1),
        A2.reshape((B + 1) * N, _CH2),
        B, M, N).reshape(B, M, _CH)

    OC = Wagg.shape[1]
    out = pl.pallas_call(
        _fin_body,
        grid=(B, M // _MB2),
        in_specs=[
            pl.BlockSpec((1, _MB2, _CH), lambda b, mi: (b, mi, 0)),
            pl.BlockSpec((1, _MB2, D), lambda b, mi: (b, mi, 0)),
            pl.BlockSpec((D, _CH), lambda b, mi: (0, 0)),
            pl.BlockSpec((_CH, OC), lambda b, mi: (0, 0)),
            pl.BlockSpec((1, OC), lambda b, mi: (0, 0)),
        ],
        out_specs=pl.BlockSpec((1, _MB2, OC), lambda b, mi: (b, mi, 0)),
        out_shape=jax.ShapeDtypeStruct((B, M, OC), jnp.float32),
    )(mx, centers, W[0:3], Wagg, bagg[None])
    return out


# speedup vs baseline: 48.8562x; 1.0350x over previous
import functools

import jax
import jax.numpy as jnp
from jax import lax
from jax.experimental import pallas as pl
from jax.experimental.pallas import tpu as pltpu
from jax.experimental.pallas import tpu_sc as plsc

_LO1, _HI1 = 1.0, 2.25
_LO2, _HI2 = 2.25, 9.0
_K1, _K2 = 16, 32
_CH1, _CH2 = 64, 128
_CH = _CH1 + _CH2
_NEG = -3.0e38
_MB2 = 512


def _mm_body(p_ref, f_ref, w_ref, b_ref, a1_ref, a2_ref):
    bi = pl.program_id(0)
    last = pl.num_programs(0) - 1

    @pl.when(bi != last)
    def _():
        w = w_ref[...]
        a = (jnp.dot(p_ref[0], w[:3], preferred_element_type=jnp.float32)
             + jnp.dot(f_ref[0], w[3:], preferred_element_type=jnp.float32)
             + b_ref[...])
        a1_ref[0] = jnp.concatenate([a[:, :_CH1], a[:, :_CH1]], axis=1)
        a2_ref[0] = a[:, _CH1:]

    @pl.when(bi == last)
    def _():
        a1_ref[0] = jnp.full(a1_ref.shape[1:], _NEG, jnp.float32)
        a2_ref[0] = jnp.full(a2_ref.shape[1:], _NEG, jnp.float32)


def _fin_body(mx_ref, cen_ref, w3_ref, wagg_ref, bagg_ref, o_ref):
    c = jnp.dot(cen_ref[0], w3_ref[...], preferred_element_type=jnp.float32)
    f = jax.nn.relu(mx_ref[0] - c)
    o_ref[0] = jax.nn.relu(
        jnp.dot(f, wagg_ref[...], preferred_element_type=jnp.float32)
        + bagg_ref[...])


def _sc_select_max(dist2, a1t, a2t, B, M, N):
    NC, NS, L = 2, 16, 16
    NW = NC * NS
    RPW = (B * M) // NW
    PAD = B * N
    NV = N // L
    _CVS = 32
    mesh = plsc.VectorSubcoreMesh(core_axis_name="c", subcore_axis_name="s",
                                  num_cores=NC, num_subcores=NS)

    @functools.partial(
        pl.kernel, mesh=mesh,
        compiler_params=pltpu.CompilerParams(needs_layout_passes=False),
        out_type=jax.ShapeDtypeStruct((B * M, _CH), jnp.float32),
        scratch_types=[
            pltpu.VMEM((N,), jnp.float32),
            pltpu.VMEM((N,), jnp.float32),
            pltpu.VMEM((_K1,), jnp.int32),
            pltpu.VMEM((_K1,), jnp.int32),
            pltpu.VMEM((_K2,), jnp.int32),
            pltpu.VMEM((_K2,), jnp.int32),
            pltpu.VMEM((_K1, 2 * _CH1), jnp.float32),
            pltpu.VMEM((_K1, 2 * _CH1), jnp.float32),
            pltpu.VMEM((_K2, _CH2), jnp.float32),
            pltpu.VMEM((_K2, _CH2), jnp.float32),
            pltpu.VMEM((2, _CH), jnp.float32),
            pltpu.SemaphoreType.DMA,
            pltpu.SemaphoreType.DMA,
            pltpu.SemaphoreType.DMA,
            pltpu.SemaphoreType.DMA,
            pltpu.SemaphoreType.DMA,
            pltpu.SemaphoreType.DMA,
            pltpu.SemaphoreType.DMA,
        ],
    )
    def sck(dist_hbm, a1_hbm, a2_hbm, out_hbm,
            dbufA, dbufB, idx1A, idx1B, idx2A, idx2B,
            rows1A, rows1B, rows2A, rows2B, obuf2,
            sdA, sdB, g1A, g2A, g1B, g2B, so):
        wid = lax.axis_index("s") * NC + lax.axis_index("c")
        base = wid * RPW
        BM = B * M
        iota = lax.broadcasted_iota(jnp.int32, (L,), 0)
        padv = jnp.full((L,), PAD, jnp.int32)
        zero = jnp.zeros((L,), jnp.int32)

        def scan_row(g, dbuf, idx1, idx2):
            b = g // M
            boff = b * N
            idx1[...] = padv
            idx2[pl.ds(0, L)] = padv
            idx2[pl.ds(L, L)] = padv

            lstep = jnp.full((L,), L, jnp.int32)
            cstep = jnp.full((L,), _CVS * L, jnp.int32)

            def _full_chunk(c, tt):
                t1, t2, lb = tt

                @plsc.parallel_loop(0, _CVS, unroll=4, carry=(t1, t2, lb))
                def _scan(jj, c2):
                    t1c, t2c, lidx = c2
                    j = c * _CVS + jj
                    d = dbuf[pl.ds(j * L, L)]
                    ge1 = d >= _LO1
                    ge2 = d >= _HI1
                    lt9 = d < _HI2
                    p1 = ge1 & (~ge2)
                    p2 = ge2 & lt9
                    i1 = p1.astype(jnp.int32)
                    i2 = p2.astype(jnp.int32)
                    r1 = t1c + plsc.cumsum(i1) - i1
                    r2 = t2c + plsc.cumsum(i2) - i2
                    k1 = p1 & (r1 < _K1)
                    k2 = p2 & (r2 < _K2)
                    plsc.store_scatter(idx1, [r1], lidx, mask=k1)
                    plsc.store_scatter(idx2, [r2], lidx, mask=k2)
                    t1c = t1c + plsc.all_reduce_population_count(p1)
                    t2c = t2c + plsc.all_reduce_population_count(p2)
                    return t1c, t2c, lidx + lstep

                return _scan

            def _s2_chunk(c, tt):
                t1, t2, lb = tt

                @plsc.parallel_loop(0, _CVS, unroll=8, carry=(t2, lb))
                def _scan(jj, c2):
                    t2c, lidx = c2
                    j = c * _CVS + jj
                    d = dbuf[pl.ds(j * L, L)]
                    p2 = (d >= _HI1) & (d < _HI2)
                    i2 = p2.astype(jnp.int32)
                    r2 = t2c + plsc.cumsum(i2) - i2
                    k2 = p2 & (r2 < _K2)
                    plsc.store_scatter(idx2, [r2], lidx, mask=k2)
                    return t2c + plsc.all_reduce_population_count(p2), lidx + lstep

                return t1, _scan[0], _scan[1]

            def chunk_body(c, tt):
                s = jnp.max(tt[0] * 65536 + tt[1], axis=0)
                cnt1 = s // 65536
                cnt2 = s - cnt1 * 65536
                return lax.cond(
                    cnt1 >= _K1,
                    lambda tt2: lax.cond(
                        cnt2 >= _K2,
                        lambda tt3: (tt3[0], tt3[1], tt3[2] + cstep),
                        lambda tt3: _s2_chunk(c, tt3),
                        tt2),
                    lambda tt2: _full_chunk(c, tt2),
                    tt)

            lax.fori_loop(0, NV // _CVS, chunk_body,
                          (zero, zero, iota + boff))

        def max_row(rows1, rows2, slot):
            for c in range(_CH1 // L):
                acc = rows1[0, pl.ds(c * L, L)]
                for r in range(1, _K1):
                    acc = jnp.maximum(acc, rows1[r, pl.ds(c * L, L)])
                obuf2[slot, pl.ds(c * L, L)] = acc
            for c in range(_CH2 // L):
                acc = rows2[0, pl.ds(c * L, L)]
                for r in range(1, _K2):
                    acc = jnp.maximum(acc, rows2[r, pl.ds(c * L, L)])
                obuf2[slot, pl.ds(_CH1 + c * L, L)] = acc

        pltpu.async_copy(dist_hbm.at[base], dbufA, sdA)
        pltpu.async_copy(dist_hbm.at[base + 1], dbufB, sdB)

        def pair_body(p, carry):
            ga = base + 2 * p
            gb = ga + 1
            pltpu.make_async_copy(dist_hbm.at[ga], dbufA, sdA).wait()
            scan_row(ga, dbufA, idx1A, idx2A)
            cpA1 = pltpu.async_copy(a1_hbm.at[idx1A], rows1A, g1A)
            cpA2 = pltpu.async_copy(a2_hbm.at[idx2A], rows2A, g2A)
            pltpu.async_copy(dist_hbm.at[jnp.minimum(ga + 2, BM - 1)],
                             dbufA, sdA)
            pltpu.make_async_copy(dist_hbm.at[gb], dbufB, sdB).wait()
            scan_row(gb, dbufB, idx1B, idx2B)
            cpB1 = pltpu.async_copy(a1_hbm.at[idx1B], rows1B, g1B)
            cpB2 = pltpu.async_copy(a2_hbm.at[idx2B], rows2B, g2B)
            pltpu.async_copy(dist_hbm.at[jnp.minimum(gb + 2, BM - 1)],
                             dbufB, sdB)
            cpA1.wait()
            cpA2.wait()

            @pl.when(p > 0)
            def _():
                pltpu.make_async_copy(obuf2, out_hbm.at[pl.ds(ga - 2, 2)],
                                      so).wait()

            max_row(rows1A, rows2A, 0)
            cpB1.wait()
            cpB2.wait()
            max_row(rows1B, rows2B, 1)
            pltpu.async_copy(obuf2, out_hbm.at[pl.ds(ga, 2)], so)
            return carry

        lax.fori_loop(0, RPW // 2, pair_body, 0)
        pltpu.make_async_copy(obuf2, out_hbm.at[pl.ds(base + RPW - 2, 2)],
                              so).wait()
        pltpu.make_async_copy(dist_hbm.at[jnp.minimum(base + RPW, BM - 1)],
                              dbufA, sdA).wait()
        pltpu.make_async_copy(dist_hbm.at[jnp.minimum(base + RPW + 1, BM - 1)],
                              dbufB, sdB).wait()

    return sck(dist2, a1t, a2t)


def kernel(positions, features, centers, distances, W0, b0, W1, b1, Wagg, bagg):
    B, N, D = positions.shape
    M = centers.shape[1]
    W = jnp.concatenate([W0, W1], axis=1)
    bcat = jnp.concatenate([b0, b1])[None]
    C = features.shape[-1]
    F = D + C

    A1, A2 = pl.pallas_call(
        _mm_body,
        grid=(B + 1,),
        in_specs=[
            pl.BlockSpec((1, N, D), lambda b: (jnp.minimum(b, B - 1), 0, 0)),
            pl.BlockSpec((1, N, C), lambda b: (jnp.minimum(b, B - 1), 0, 0)),
            pl.BlockSpec((F, _CH), lambda b: (0, 0)),
            pl.BlockSpec((1, _CH), lambda b: (0, 0)),
        ],
        out_specs=[
            pl.BlockSpec((1, N, 2 * _CH1), lambda b: (b, 0, 0)),
            pl.BlockSpec((1, N, _CH2), lambda b: (b, 0, 0)),
        ],
        out_shape=[
            jax.ShapeDtypeStruct((B + 1, N, 2 * _CH1), jnp.float32),
            jax.ShapeDtypeStruct((B + 1, N, _CH2), jnp.float32),
        ],
    )(positions, features, W, bcat)

    mx = _sc_select_max(
        distances.reshape(B * M, N),
        A1.reshape((B + 1) * N, 2 * _CH1),
        A2.reshape((B + 1) * N, _CH2),
        B, M, N).reshape(B, M, _CH)

    OC = Wagg.shape[1]
    out = pl.pallas_call(
        _fin_body,
        grid=(B, M // _MB2),
        in_specs=[
            pl.BlockSpec((1, _MB2, _CH), lambda b, mi: (b, mi, 0)),
            pl.BlockSpec((1, _MB2, D), lambda b, mi: (b, mi, 0)),
            pl.BlockSpec((D, _CH), lambda b, mi: (0, 0)),
            pl.BlockSpec((_CH, OC), lambda b, mi: (0, 0)),
            pl.BlockSpec((1, OC), lambda b, mi: (0, 0)),
        ],
        out_specs=pl.BlockSpec((1, _MB2, OC), lambda b, mi: (b, mi, 0)),
        out_shape=jax.ShapeDtypeStruct((B, M, OC), jnp.float32),
    )(mx, centers, W[0:3], Wagg, bagg[None])
    return out


# speedup vs baseline: 50.0442x; 1.0243x over previous
import functools

import jax
import jax.numpy as jnp
from jax import lax
from jax.experimental import pallas as pl
from jax.experimental.pallas import tpu as pltpu
from jax.experimental.pallas import tpu_sc as plsc

_LO1, _HI1 = 1.0, 2.25
_LO2, _HI2 = 2.25, 9.0
_K1, _K2 = 16, 32
_CH1, _CH2 = 64, 128
_CH = _CH1 + _CH2
_NEG = -3.0e38
_MB2 = 512


def _mm_body(p_ref, f_ref, w_ref, b_ref, a1_ref, a2_ref):
    bi = pl.program_id(0)
    last = pl.num_programs(0) - 1

    @pl.when(bi != last)
    def _():
        w = w_ref[...]
        a = (jnp.dot(p_ref[0], w[:3], preferred_element_type=jnp.float32)
             + jnp.dot(f_ref[0], w[3:], preferred_element_type=jnp.float32)
             + b_ref[...])
        a1_ref[0] = jnp.concatenate([a[:, :_CH1], a[:, :_CH1]], axis=1)
        a2_ref[0] = a[:, _CH1:]

    @pl.when(bi == last)
    def _():
        a1_ref[0] = jnp.full(a1_ref.shape[1:], _NEG, jnp.float32)
        a2_ref[0] = jnp.full(a2_ref.shape[1:], _NEG, jnp.float32)


def _fin_body(mx_ref, cen_ref, w3_ref, wagg_ref, bagg_ref, o_ref):
    c = jnp.dot(cen_ref[0], w3_ref[...], preferred_element_type=jnp.float32)
    f = jax.nn.relu(mx_ref[0] - c)
    o_ref[0] = jax.nn.relu(
        jnp.dot(f, wagg_ref[...], preferred_element_type=jnp.float32)
        + bagg_ref[...])


def _sc_select_max(dist2, a1t, a2t, B, M, N):
    NC, NS, L = 2, 16, 16
    NW = NC * NS
    RPW = (B * M) // NW
    PAD = B * N
    NV = N // L
    _CVS = 64
    mesh = plsc.VectorSubcoreMesh(core_axis_name="c", subcore_axis_name="s",
                                  num_cores=NC, num_subcores=NS)

    @functools.partial(
        pl.kernel, mesh=mesh,
        compiler_params=pltpu.CompilerParams(needs_layout_passes=False),
        out_type=jax.ShapeDtypeStruct((B * M, _CH), jnp.float32),
        scratch_types=[
            pltpu.VMEM((N,), jnp.float32),
            pltpu.VMEM((N,), jnp.float32),
            pltpu.VMEM((_K1,), jnp.int32),
            pltpu.VMEM((_K1,), jnp.int32),
            pltpu.VMEM((_K2,), jnp.int32),
            pltpu.VMEM((_K2,), jnp.int32),
            pltpu.VMEM((_K1, 2 * _CH1), jnp.float32),
            pltpu.VMEM((_K1, 2 * _CH1), jnp.float32),
            pltpu.VMEM((_K2, _CH2), jnp.float32),
            pltpu.VMEM((_K2, _CH2), jnp.float32),
            pltpu.VMEM((2, _CH), jnp.float32),
            pltpu.SemaphoreType.DMA,
            pltpu.SemaphoreType.DMA,
            pltpu.SemaphoreType.DMA,
            pltpu.SemaphoreType.DMA,
            pltpu.SemaphoreType.DMA,
            pltpu.SemaphoreType.DMA,
            pltpu.SemaphoreType.DMA,
        ],
    )
    def sck(dist_hbm, a1_hbm, a2_hbm, out_hbm,
            dbufA, dbufB, idx1A, idx1B, idx2A, idx2B,
            rows1A, rows1B, rows2A, rows2B, obuf2,
            sdA, sdB, g1A, g2A, g1B, g2B, so):
        wid = lax.axis_index("s") * NC + lax.axis_index("c")
        base = wid * RPW
        BM = B * M
        iota = lax.broadcasted_iota(jnp.int32, (L,), 0)
        padv = jnp.full((L,), PAD, jnp.int32)
        zero = jnp.zeros((L,), jnp.int32)

        def scan_row(g, dbuf, idx1, idx2):
            b = g // M
            boff = b * N
            idx1[...] = padv
            idx2[pl.ds(0, L)] = padv
            idx2[pl.ds(L, L)] = padv

            lstep = jnp.full((L,), L, jnp.int32)
            cstep = jnp.full((L,), _CVS * L, jnp.int32)

            def _full_chunk(c, tt):
                t1, t2, lb = tt

                @plsc.parallel_loop(0, _CVS, unroll=4, carry=(t1, t2, lb))
                def _scan(jj, c2):
                    t1c, t2c, lidx = c2
                    j = c * _CVS + jj
                    d = dbuf[pl.ds(j * L, L)]
                    ge1 = d >= _LO1
                    ge2 = d >= _HI1
                    lt9 = d < _HI2
                    p1 = ge1 & (~ge2)
                    p2 = ge2 & lt9
                    i1 = p1.astype(jnp.int32)
                    i2 = p2.astype(jnp.int32)
                    r1 = t1c + plsc.cumsum(i1) - i1
                    r2 = t2c + plsc.cumsum(i2) - i2
                    k1 = p1 & (r1 < _K1)
                    k2 = p2 & (r2 < _K2)
                    plsc.store_scatter(idx1, [r1], lidx, mask=k1)
                    plsc.store_scatter(idx2, [r2], lidx, mask=k2)
                    t1c = t1c + plsc.all_reduce_population_count(p1)
                    t2c = t2c + plsc.all_reduce_population_count(p2)
                    return t1c, t2c, lidx + lstep

                return _scan

            def _s2_chunk(c, tt):
                t1, t2, lb = tt

                @plsc.parallel_loop(0, _CVS, unroll=8, carry=(t2, lb))
                def _scan(jj, c2):
                    t2c, lidx = c2
                    j = c * _CVS + jj
                    d = dbuf[pl.ds(j * L, L)]
                    p2 = (d >= _HI1) & (d < _HI2)
                    i2 = p2.astype(jnp.int32)
                    r2 = t2c + plsc.cumsum(i2) - i2
                    k2 = p2 & (r2 < _K2)
                    plsc.store_scatter(idx2, [r2], lidx, mask=k2)
                    return t2c + plsc.all_reduce_population_count(p2), lidx + lstep

                return t1, _scan[0], _scan[1]

            def chunk_body(c, tt):
                s = jnp.max(tt[0] * 65536 + tt[1], axis=0)
                cnt1 = s // 65536
                cnt2 = s - cnt1 * 65536
                return lax.cond(
                    cnt1 >= _K1,
                    lambda tt2: lax.cond(
                        cnt2 >= _K2,
                        lambda tt3: (tt3[0], tt3[1], tt3[2] + cstep),
                        lambda tt3: _s2_chunk(c, tt3),
                        tt2),
                    lambda tt2: _full_chunk(c, tt2),
                    tt)

            lax.fori_loop(0, NV // _CVS, chunk_body,
                          (zero, zero, iota + boff))

        def max_row(rows1, rows2, slot):
            for c in range(_CH1 // L):
                acc = rows1[0, pl.ds(c * L, L)]
                for r in range(1, _K1):
                    acc = jnp.maximum(acc, rows1[r, pl.ds(c * L, L)])
                obuf2[slot, pl.ds(c * L, L)] = acc
            for c in range(_CH2 // L):
                acc = rows2[0, pl.ds(c * L, L)]
                for r in range(1, _K2):
                    acc = jnp.maximum(acc, rows2[r, pl.ds(c * L, L)])
                obuf2[slot, pl.ds(_CH1 + c * L, L)] = acc

        pltpu.async_copy(dist_hbm.at[base], dbufA, sdA)
        pltpu.async_copy(dist_hbm.at[base + 1], dbufB, sdB)

        def pair_body(p, carry):
            ga = base + 2 * p
            gb = ga + 1
            pltpu.make_async_copy(dist_hbm.at[ga], dbufA, sdA).wait()
            scan_row(ga, dbufA, idx1A, idx2A)
            cpA1 = pltpu.async_copy(a1_hbm.at[idx1A], rows1A, g1A)
            cpA2 = pltpu.async_copy(a2_hbm.at[idx2A], rows2A, g2A)
            pltpu.async_copy(dist_hbm.at[jnp.minimum(ga + 2, BM - 1)],
                             dbufA, sdA)
            pltpu.make_async_copy(dist_hbm.at[gb], dbufB, sdB).wait()
            scan_row(gb, dbufB, idx1B, idx2B)
            cpB1 = pltpu.async_copy(a1_hbm.at[idx1B], rows1B, g1B)
            cpB2 = pltpu.async_copy(a2_hbm.at[idx2B], rows2B, g2B)
            pltpu.async_copy(dist_hbm.at[jnp.minimum(gb + 2, BM - 1)],
                             dbufB, sdB)
            cpA1.wait()
            cpA2.wait()

            @pl.when(p > 0)
            def _():
                pltpu.make_async_copy(obuf2, out_hbm.at[pl.ds(ga - 2, 2)],
                                      so).wait()

            max_row(rows1A, rows2A, 0)
            cpB1.wait()
            cpB2.wait()
            max_row(rows1B, rows2B, 1)
            pltpu.async_copy(obuf2, out_hbm.at[pl.ds(ga, 2)], so)
            return carry

        lax.fori_loop(0, RPW // 2, pair_body, 0)
        pltpu.make_async_copy(obuf2, out_hbm.at[pl.ds(base + RPW - 2, 2)],
                              so).wait()
        pltpu.make_async_copy(dist_hbm.at[jnp.minimum(base + RPW, BM - 1)],
                              dbufA, sdA).wait()
        pltpu.make_async_copy(dist_hbm.at[jnp.minimum(base + RPW + 1, BM - 1)],
                              dbufB, sdB).wait()

    return sck(dist2, a1t, a2t)


def kernel(positions, features, centers, distances, W0, b0, W1, b1, Wagg, bagg):
    B, N, D = positions.shape
    M = centers.shape[1]
    W = jnp.concatenate([W0, W1], axis=1)
    bcat = jnp.concatenate([b0, b1])[None]
    C = features.shape[-1]
    F = D + C

    A1, A2 = pl.pallas_call(
        _mm_body,
        grid=(B + 1,),
        in_specs=[
            pl.BlockSpec((1, N, D), lambda b: (jnp.minimum(b, B - 1), 0, 0)),
            pl.BlockSpec((1, N, C), lambda b: (jnp.minimum(b, B - 1), 0, 0)),
            pl.BlockSpec((F, _CH), lambda b: (0, 0)),
            pl.BlockSpec((1, _CH), lambda b: (0, 0)),
        ],
        out_specs=[
            pl.BlockSpec((1, N, 2 * _CH1), lambda b: (b, 0, 0)),
            pl.BlockSpec((1, N, _CH2), lambda b: (b, 0, 0)),
        ],
        out_shape=[
            jax.ShapeDtypeStruct((B + 1, N, 2 * _CH1), jnp.float32),
            jax.ShapeDtypeStruct((B + 1, N, _CH2), jnp.float32),
        ],
    )(positions, features, W, bcat)

    mx = _sc_select_max(
        distances.reshape(B * M, N),
        A1.reshape((B + 1) * N, 2 * _CH1),
        A2.reshape((B + 1) * N, _CH2),
        B, M, N).reshape(B, M, _CH)

    OC = Wagg.shape[1]
    out = pl.pallas_call(
        _fin_body,
        grid=(B, M // _MB2),
        in_specs=[
            pl.BlockSpec((1, _MB2, _CH), lambda b, mi: (b, mi, 0)),
            pl.BlockSpec((1, _MB2, D), lambda b, mi: (b, mi, 0)),
            pl.BlockSpec((D, _CH), lambda b, mi: (0, 0)),
            pl.BlockSpec((_CH, OC), lambda b, mi: (0, 0)),
            pl.BlockSpec((1, OC), lambda b, mi: (0, 0)),
        ],
        out_specs=pl.BlockSpec((1, _MB2, OC), lambda b, mi: (b, mi, 0)),
        out_shape=jax.ShapeDtypeStruct((B, M, OC), jnp.float32),
    )(mx, centers, W[0:3], Wagg, bagg[None])
    return out
